# baseline (speedup 1.0000x reference)
import functools

import jax
import jax.numpy as jnp
from jax.experimental import pallas as pl

_NUM_TABLES = 26
_BATCH = 4096
_K_HASH = 128
_EMB_DIM = 64
_M_HASH = 1000000
_BT = 512


def _fused_body(xd, xi, ha, hb, w1, b1, w2, b2,
                bw1, bb1, bw2, bb2, bw3, bb3,
                tw1, tb1, tw2, tb2, tw3t, tb3, out):
    f32 = jnp.float32
    dot = functools.partial(jnp.dot, preferred_element_type=f32)

    h = jnp.maximum(dot(xd[...], bw1[...]) + bb1[...], 0.0)
    h = jnp.maximum(dot(h, bw2[...]) + bb2[...], 0.0)
    h = jnp.maximum(dot(h, bw3[...]) + bb3[...], 0.0)

    acc = dot(h, tw1[0:_EMB_DIM, :]) + tb1[...]

    scale = f32(2.0 / (_M_HASH - 1))
    for t in range(_NUM_TABLES):
        idx = xi[t, :].astype(jnp.uint32)
        a = ha[t, :].astype(jnp.uint32)
        b = hb[t, :].astype(jnp.uint32)
        hv = (idx[:, None] * a[None, :] + b[None, :]) % jnp.uint32(_M_HASH)
        enc = hv.astype(f32) * scale - 1.0
        e1 = jnp.maximum(dot(enc, w1[t]) + b1[t, :][None, :], 0.0)
        emb = dot(e1, w2[t]) + b2[t, :][None, :]
        lo = _EMB_DIM * (t + 1)
        acc = acc + dot(emb, tw1[lo:lo + _EMB_DIM, :])

    z = jnp.maximum(acc, 0.0)
    z = jnp.maximum(dot(z, tw2[...]) + tb2[...], 0.0)
    o = jnp.sum(z * tw3t[...], axis=1, keepdims=True) + tb3[...]
    out[...] = jax.nn.sigmoid(o)


def _make_call(interpret=False):
    bt = _BT
    grid = (_BATCH // bt,)

    def batch_spec(shape):
        return pl.BlockSpec(shape, lambda i: (i, 0))

    def const_spec(shape):
        nd = len(shape)
        if nd == 2:
            return pl.BlockSpec(shape, lambda i: (0, 0))
        return pl.BlockSpec(shape, lambda i: (0, 0, 0))

    in_specs = [
        batch_spec((bt, 13)),
        pl.BlockSpec((_NUM_TABLES, bt), lambda i: (0, i)),
        const_spec((_NUM_TABLES, _K_HASH)),
        const_spec((_NUM_TABLES, _K_HASH)),
        const_spec((_NUM_TABLES, _K_HASH, 256)),
        const_spec((_NUM_TABLES, 256)),
        const_spec((_NUM_TABLES, 256, _EMB_DIM)),
        const_spec((_NUM_TABLES, _EMB_DIM)),
        const_spec((13, 512)),
        const_spec((1, 512)),
        const_spec((512, 256)),
        const_spec((1, 256)),
        const_spec((256, 64)),
        const_spec((1, 64)),
        const_spec((27 * _EMB_DIM, 512)),
        const_spec((1, 512)),
        const_spec((512, 256)),
        const_spec((1, 256)),
        const_spec((1, 256)),
        const_spec((1, 1)),
    ]
    return pl.pallas_call(
        _fused_body,
        grid=grid,
        in_specs=in_specs,
        out_specs=pl.BlockSpec((bt, 1), lambda i: (i, 0)),
        out_shape=jax.ShapeDtypeStruct((_BATCH, 1), jnp.float32),
        interpret=interpret,
    )


def kernel(x_dense, x_indices, hash_a, hash_b, dec_W1, dec_b1, dec_W2,
           dec_b2, bot_W1, bot_b1, bot_W2, bot_b2, bot_W3, bot_b3,
           top_W1, top_b1, top_W2, top_b2, top_W3, top_b3):
    call = _make_call()
    return call(
        x_dense, x_indices, hash_a, hash_b,
        dec_W1, dec_b1, dec_W2, dec_b2,
        bot_W1, bot_b1.reshape(1, -1),
        bot_W2, bot_b2.reshape(1, -1),
        bot_W3, bot_b3.reshape(1, -1),
        top_W1, top_b1.reshape(1, -1),
        top_W2, top_b2.reshape(1, -1),
        top_W3.reshape(1, -1), top_b3.reshape(1, -1),
    )


# speedup vs baseline: 1.4303x; 1.4303x over previous
import functools

import jax
import jax.numpy as jnp
from jax.experimental import pallas as pl
from jax.experimental.pallas import tpu as pltpu

_NUM_TABLES = 26
_BATCH = 4096
_K_HASH = 128
_EMB_DIM = 64
_M_HASH = 1000000
_BT = 512


def _fused_body(xd, xi, ha, hb, w1, b1, w2, b2,
                bw1, bb1, bw2, bb2, bw3, bb3,
                tw1, tb1, tw2, tb2, tw3t, tb3, out, zbuf):
    f32 = jnp.float32
    dot = functools.partial(jnp.dot, preferred_element_type=f32)

    h = jnp.maximum(dot(xd[...], bw1[...]) + bb1[...], 0.0)
    h = jnp.maximum(dot(h, bw2[...]) + bb2[...], 0.0)
    h = jnp.maximum(dot(h, bw3[...]) + bb3[...], 0.0)
    zbuf[:, 0:_EMB_DIM] = h

    scale = f32(2.0 / (_M_HASH - 1))
    for t in range(_NUM_TABLES):
        idx = xi[t, :].astype(jnp.uint32)
        a = ha[t, :].astype(jnp.uint32)
        b = hb[t, :].astype(jnp.uint32)
        hv = (idx[:, None] * a[None, :] + b[None, :]) % jnp.uint32(_M_HASH)
        enc = hv.astype(f32) * scale - 1.0
        e1 = jnp.maximum(dot(enc, w1[t]) + b1[t, :][None, :], 0.0)
        emb = dot(e1, w2[t]) + b2[t, :][None, :]
        lo = _EMB_DIM * (t + 1)
        zbuf[:, lo:lo + _EMB_DIM] = emb

    z = jnp.maximum(dot(zbuf[...], tw1[...]) + tb1[...], 0.0)
    z = jnp.maximum(dot(z, tw2[...]) + tb2[...], 0.0)
    o = jnp.sum(z * tw3t[...], axis=1, keepdims=True) + tb3[...]
    out[...] = jax.nn.sigmoid(o)


def _make_call(interpret=False):
    bt = _BT
    grid = (_BATCH // bt,)

    def batch_spec(shape):
        return pl.BlockSpec(shape, lambda i: (i, 0))

    def const_spec(shape):
        nd = len(shape)
        if nd == 2:
            return pl.BlockSpec(shape, lambda i: (0, 0))
        return pl.BlockSpec(shape, lambda i: (0, 0, 0))

    in_specs = [
        batch_spec((bt, 13)),
        pl.BlockSpec((_NUM_TABLES, bt), lambda i: (0, i)),
        const_spec((_NUM_TABLES, _K_HASH)),
        const_spec((_NUM_TABLES, _K_HASH)),
        const_spec((_NUM_TABLES, _K_HASH, 256)),
        const_spec((_NUM_TABLES, 256)),
        const_spec((_NUM_TABLES, 256, _EMB_DIM)),
        const_spec((_NUM_TABLES, _EMB_DIM)),
        const_spec((13, 512)),
        const_spec((1, 512)),
        const_spec((512, 256)),
        const_spec((1, 256)),
        const_spec((256, 64)),
        const_spec((1, 64)),
        const_spec((27 * _EMB_DIM, 512)),
        const_spec((1, 512)),
        const_spec((512, 256)),
        const_spec((1, 256)),
        const_spec((1, 256)),
        const_spec((1, 1)),
    ]
    return pl.pallas_call(
        _fused_body,
        grid=grid,
        in_specs=in_specs,
        out_specs=pl.BlockSpec((bt, 1), lambda i: (i, 0)),
        out_shape=jax.ShapeDtypeStruct((_BATCH, 1), jnp.float32),
        scratch_shapes=[pltpu.VMEM((bt, 27 * _EMB_DIM), jnp.float32)],
        interpret=interpret,
    )


def kernel(x_dense, x_indices, hash_a, hash_b, dec_W1, dec_b1, dec_W2,
           dec_b2, bot_W1, bot_b1, bot_W2, bot_b2, bot_W3, bot_b3,
           top_W1, top_b1, top_W2, top_b2, top_W3, top_b3):
    call = _make_call()
    return call(
        x_dense, x_indices, hash_a, hash_b,
        dec_W1, dec_b1, dec_W2, dec_b2,
        bot_W1, bot_b1.reshape(1, -1),
        bot_W2, bot_b2.reshape(1, -1),
        bot_W3, bot_b3.reshape(1, -1),
        top_W1, top_b1.reshape(1, -1),
        top_W2, top_b2.reshape(1, -1),
        top_W3.reshape(1, -1), top_b3.reshape(1, -1),
    )
